# Initial kernel scaffold; baseline (speedup 1.0000x reference)
import functools

import jax
import jax.numpy as jnp
from jax import lax
from jax.experimental import pallas as pl
from jax.experimental.pallas import tpu as pltpu
from jax.experimental.pallas import tpu_sc as plsc

N = 65536
B = 64
SPN1 = 17
NNZ = N * SPN1

NUM_WORKERS = 32
COLS_PER_WORKER = B // NUM_WORKERS
CHUNK_ROOTS = 1024
CHUNK_ENTRIES = CHUNK_ROOTS * SPN1
NUM_CHUNKS = N // CHUNK_ROOTS
LANES = 16


def _sc_body(xt_hbm, rows_hbm, cols_hbm, vals_hbm, out_hbm,
             acc, xv, rv, cv, vv):
    cid = lax.axis_index("c")
    sid = lax.axis_index("s")
    wid = sid * 2 + cid

    for col_i in range(COLS_PER_WORKER):
        col = wid + NUM_WORKERS * col_i

        def _zero(i, _):
            acc[pl.ds(i * LANES, LANES)] = jnp.zeros((LANES,), jnp.float32)
            return 0
        lax.fori_loop(0, N // LANES, _zero, 0, unroll=8)

        def _chunk(j, _):
            root0 = j * CHUNK_ROOTS
            e0 = j * CHUNK_ENTRIES
            pltpu.sync_copy(xt_hbm.at[col, pl.ds(root0, CHUNK_ROOTS)], xv)
            pltpu.sync_copy(rows_hbm.at[pl.ds(e0, CHUNK_ENTRIES)], rv)
            pltpu.sync_copy(cols_hbm.at[pl.ds(e0, CHUNK_ENTRIES)], cv)
            pltpu.sync_copy(vals_hbm.at[pl.ds(e0, CHUNK_ENTRIES)], vv)

            def _inner(i, _):
                idx = rv[pl.ds(i * LANES, LANES)]
                ci = cv[pl.ds(i * LANES, LANES)] - root0
                v = vv[pl.ds(i * LANES, LANES)]
                xx = plsc.load_gather(xv, [ci])
                plsc.addupdate_scatter(acc, [idx], v * xx)
                return 0
            lax.fori_loop(0, CHUNK_ENTRIES // LANES, _inner, 0)
            return 0
        lax.fori_loop(0, NUM_CHUNKS, _chunk, 0)

        pltpu.sync_copy(acc, out_hbm.at[col, :])


def _sc_scatter(xt, rows, cols, vals):
    mesh = plsc.VectorSubcoreMesh(core_axis_name="c", subcore_axis_name="s")
    f = pl.kernel(
        _sc_body,
        out_type=jax.ShapeDtypeStruct((B, N), jnp.float32),
        mesh=mesh,
        scratch_types=[
            pltpu.VMEM((N,), jnp.float32),
            pltpu.VMEM((CHUNK_ROOTS,), jnp.float32),
            pltpu.VMEM((CHUNK_ENTRIES,), jnp.int32),
            pltpu.VMEM((CHUNK_ENTRIES,), jnp.int32),
            pltpu.VMEM((CHUNK_ENTRIES,), jnp.float32),
        ],
    )
    return f(xt, rows, cols, vals)


def kernel(x, weight_vals, weight_rows, weight_cols):
    xt = x.T
    out_t = _sc_scatter(xt,
                        weight_rows.astype(jnp.int32),
                        weight_cols.astype(jnp.int32),
                        weight_vals)
    return out_t.T


# speedup vs baseline: 3.1030x; 3.1030x over previous
import functools

import jax
import jax.numpy as jnp
from jax import lax
from jax.experimental import pallas as pl
from jax.experimental.pallas import tpu as pltpu
from jax.experimental.pallas import tpu_sc as plsc

N = 65536
B = 64
SPN1 = 17
NNZ = N * SPN1

NUM_WORKERS = 32
COLS_PER_WORKER = B // NUM_WORKERS
CHUNK_ROOTS = 1024
CHUNK_ENTRIES = CHUNK_ROOTS * SPN1
NUM_CHUNKS = N // CHUNK_ROOTS
LANES = 16


def _sc_body(xt_hbm, rows_hbm, cols_hbm, vals_hbm, out_hbm,
             acc, xv, rv, cv, vv):
    cid = lax.axis_index("c")
    sid = lax.axis_index("s")
    wid = sid * 2 + cid

    for col_i in range(COLS_PER_WORKER):
        col = wid + NUM_WORKERS * col_i

        def _zero(i, _):
            acc[pl.ds(i * LANES, LANES)] = jnp.zeros((LANES,), jnp.float32)
            return 0
        lax.fori_loop(0, N // LANES, _zero, 0, unroll=8)

        def _chunk(j, _):
            root0 = j * CHUNK_ROOTS
            e0 = j * CHUNK_ENTRIES
            pltpu.sync_copy(xt_hbm.at[col, pl.ds(root0, CHUNK_ROOTS)], xv)
            pltpu.sync_copy(rows_hbm.at[pl.ds(e0, CHUNK_ENTRIES)], rv)
            pltpu.sync_copy(cols_hbm.at[pl.ds(e0, CHUNK_ENTRIES)], cv)
            pltpu.sync_copy(vals_hbm.at[pl.ds(e0, CHUNK_ENTRIES)], vv)

            def _inner(i, _):
                idx = rv[pl.ds(i * LANES, LANES)]
                ci = cv[pl.ds(i * LANES, LANES)] - root0
                v = vv[pl.ds(i * LANES, LANES)]
                xx = plsc.load_gather(xv, [ci])
                plsc.addupdate_scatter(acc, [idx], v * xx)
                return 0
            lax.fori_loop(0, CHUNK_ENTRIES // LANES, _inner, 0)
            return 0
        lax.fori_loop(0, NUM_CHUNKS, _chunk, 0)

        pltpu.sync_copy(acc, out_hbm.at[col, :])


def _sc_scatter(xt, rows, cols, vals):
    mesh = plsc.VectorSubcoreMesh(core_axis_name="c", subcore_axis_name="s")
    f = pl.kernel(
        _sc_body,
        out_type=jax.ShapeDtypeStruct((B, N), jnp.float32),
        mesh=mesh,
        scratch_types=[
            pltpu.VMEM((N,), jnp.float32),
            pltpu.VMEM((CHUNK_ROOTS,), jnp.float32),
            pltpu.VMEM((CHUNK_ENTRIES,), jnp.int32),
            pltpu.VMEM((CHUNK_ENTRIES,), jnp.int32),
            pltpu.VMEM((CHUNK_ENTRIES,), jnp.float32),
        ],
        compiler_params=pltpu.CompilerParams(needs_layout_passes=False),
    )
    return f(xt, rows, cols, vals)


def kernel(x, weight_vals, weight_rows, weight_cols):
    xt = x.T
    out_t = _sc_scatter(xt,
                        weight_rows.astype(jnp.int32),
                        weight_cols.astype(jnp.int32),
                        weight_vals)
    return out_t.T


# speedup vs baseline: 7.3805x; 2.3785x over previous
import functools

import jax
import jax.numpy as jnp
from jax import lax
from jax.experimental import pallas as pl
from jax.experimental.pallas import tpu as pltpu
from jax.experimental.pallas import tpu_sc as plsc

N = 65536
B = 64
SPN1 = 17

NUM_WORKERS = 32
COLS_PER_WORKER = B // NUM_WORKERS
CHUNK_ROOTS = 512
NUM_CHUNKS = N // CHUNK_ROOTS
LANES = 16
GROUPS = CHUNK_ROOTS // LANES


def _sc_body(xt_hbm, rows_hbm, vals_hbm, out_hbm,
             acc, xv, rv, vv, sem0, sem1):
    cid = lax.axis_index("c")
    sid = lax.axis_index("s")
    wid = sid * 2 + cid
    sems = (sem0, sem1)

    def issue(j, b):
        root0 = j * CHUNK_ROOTS
        pltpu.async_copy(xt_hbm.at[col, pl.ds(root0, CHUNK_ROOTS)],
                         xv.at[b], sems[b])
        pltpu.async_copy(rows_hbm.at[:, pl.ds(root0, CHUNK_ROOTS)],
                         rv.at[b], sems[b])
        pltpu.async_copy(vals_hbm.at[:, pl.ds(root0, CHUNK_ROOTS)],
                         vv.at[b], sems[b])

    def drain(j, b):
        root0 = j * CHUNK_ROOTS
        pltpu.make_async_copy(xt_hbm.at[col, pl.ds(root0, CHUNK_ROOTS)],
                              xv.at[b], sems[b]).wait()
        pltpu.make_async_copy(rows_hbm.at[:, pl.ds(root0, CHUNK_ROOTS)],
                              rv.at[b], sems[b]).wait()
        pltpu.make_async_copy(vals_hbm.at[:, pl.ds(root0, CHUNK_ROOTS)],
                              vv.at[b], sems[b]).wait()

    def compute(b):
        def group(g, _):
            base = g * LANES
            xx = xv[b, pl.ds(base, LANES)]
            for k in range(SPN1):
                idx = rv[b, k, pl.ds(base, LANES)]
                v = vv[b, k, pl.ds(base, LANES)]
                plsc.addupdate_scatter(acc, [idx], v * xx)
            return 0
        lax.fori_loop(0, GROUPS, group, 0)

    for col_i in range(COLS_PER_WORKER):
        col = wid + NUM_WORKERS * col_i

        def _zero(i, _):
            acc[pl.ds(i * LANES, LANES)] = jnp.zeros((LANES,), jnp.float32)
            return 0
        lax.fori_loop(0, N // LANES, _zero, 0, unroll=8)

        issue(0, 0)

        def _pair(jj, _):
            j0 = 2 * jj
            issue(j0 + 1, 1)
            drain(j0, 0)
            compute(0)

            @pl.when(jj < NUM_CHUNKS // 2 - 1)
            def _():
                issue(j0 + 2, 0)

            drain(j0 + 1, 1)
            compute(1)
            return 0
        lax.fori_loop(0, NUM_CHUNKS // 2, _pair, 0)

        pltpu.sync_copy(acc, out_hbm.at[col, :])


def _sc_scatter(xt, rows_km, vals_km):
    mesh = plsc.VectorSubcoreMesh(core_axis_name="c", subcore_axis_name="s")
    f = pl.kernel(
        _sc_body,
        out_type=jax.ShapeDtypeStruct((B, N), jnp.float32),
        mesh=mesh,
        scratch_types=[
            pltpu.VMEM((N,), jnp.float32),
            pltpu.VMEM((2, CHUNK_ROOTS), jnp.float32),
            pltpu.VMEM((2, SPN1, CHUNK_ROOTS), jnp.int32),
            pltpu.VMEM((2, SPN1, CHUNK_ROOTS), jnp.float32),
            pltpu.SemaphoreType.DMA,
            pltpu.SemaphoreType.DMA,
        ],
        compiler_params=pltpu.CompilerParams(needs_layout_passes=False),
    )
    return f(xt, rows_km, vals_km)


def kernel(x, weight_vals, weight_rows, weight_cols):
    del weight_cols
    xt = x.T
    rows_km = jnp.transpose(weight_rows.astype(jnp.int32).reshape(N, SPN1))
    vals_km = jnp.transpose(weight_vals.reshape(N, SPN1))
    out_t = _sc_scatter(xt, rows_km, vals_km)
    return out_t.T
